# baseline (speedup 1.0000x reference)
import functools

import jax
import jax.numpy as jnp
from jax import lax
from jax.experimental import pallas as pl
from jax.experimental.pallas import tpu as pltpu
from jax.experimental.pallas import tpu_sc as plsc

NUM_ROWS = 129
EMBED_D = 64
BATCH = 4096
HIST = 200

_NC = 2
_NS = 16
_NW = _NC * _NS
_RPW = BATCH // _NW
_L = 16
_NGRP = HIST // _L
_TAIL = HIST - _L

_mesh = plsc.VectorSubcoreMesh(core_axis_name="c", subcore_axis_name="s")


@functools.partial(
    pl.kernel,
    mesh=_mesh,
    out_type=jax.ShapeDtypeStruct((BATCH, HIST, EMBED_D), jnp.float32),
    scratch_types=[
        pltpu.VMEM((NUM_ROWS, EMBED_D), jnp.float32),
        pltpu.VMEM((HIST,), jnp.int32),
        pltpu.VMEM((HIST,), jnp.int32),
        pltpu.VMEM((HIST, EMBED_D), jnp.float32),
        pltpu.VMEM((HIST, EMBED_D), jnp.float32),
        pltpu.SemaphoreType.DMA,
        pltpu.SemaphoreType.DMA,
        pltpu.SemaphoreType.DMA,
        pltpu.SemaphoreType.DMA,
    ],
)
def _embed_gather(idx_hbm, table_hbm, out_hbm, table_v, idx0, idx1,
                  rows0, rows1, si0, si1, so0, so1):
    wid = lax.axis_index("s") * _NC + lax.axis_index("c")
    base = wid * _RPW

    pltpu.sync_copy(table_hbm, table_v)

    pltpu.async_copy(idx_hbm.at[base], idx0, si0)
    pltpu.async_copy(idx_hbm.at[base + 1], idx1, si1)

    def expand(iv_ref, rv):
        def grp(g, carry):
            iv = iv_ref[pl.ds(g * _L, _L)]
            for k in range(_L):
                row = iv[k]
                dst = g * _L + k
                for j in range(EMBED_D // _L):
                    rv[dst, pl.ds(j * _L, _L)] = table_v[row,
                                                         pl.ds(j * _L, _L)]
            return carry
        lax.fori_loop(0, _NGRP, grp, 0)
        iv = iv_ref[pl.ds(_TAIL, _L)]
        for k in range(_L):
            row = iv[k]
            for j in range(EMBED_D // _L):
                rv[_TAIL + k, pl.ds(j * _L, _L)] = table_v[row,
                                                           pl.ds(j * _L, _L)]

    def pair(i, carry):
        for b, (iv_ref, rv, si, so) in enumerate((
                (idx0, rows0, si0, so0), (idx1, rows1, si1, so1))):
            r = base + 2 * i + b
            pltpu.make_async_copy(idx_hbm.at[base], iv_ref, si).wait()

            @pl.when(i > 0)
            def _wait_out():
                pltpu.make_async_copy(rv, out_hbm.at[base], so).wait()

            expand(iv_ref, rv)
            pltpu.async_copy(rv, out_hbm.at[r], so)

            @pl.when(r + 2 < base + _RPW)
            def _prefetch_next():
                pltpu.async_copy(idx_hbm.at[r + 2], iv_ref, si)
        return carry

    lax.fori_loop(0, _RPW // 2, pair, 0)

    pltpu.make_async_copy(rows0, out_hbm.at[base], so0).wait()
    pltpu.make_async_copy(rows1, out_hbm.at[base], so1).wait()


def kernel(genres, table):
    return _embed_gather(genres.astype(jnp.int32), table)


# speedup vs baseline: 1.0002x; 1.0002x over previous
import functools

import jax
import jax.numpy as jnp
from jax import lax
from jax.experimental import pallas as pl
from jax.experimental.pallas import tpu as pltpu
from jax.experimental.pallas import tpu_sc as plsc

NUM_ROWS = 129
EMBED_D = 64
BATCH = 4096
HIST = 200

_NC = 2
_NS = 16
_NW = _NC * _NS
_RPW = BATCH // _NW
_L = 16
_NGRP = HIST // _L
_TAIL = HIST - _L

_mesh = plsc.VectorSubcoreMesh(core_axis_name="c", subcore_axis_name="s")


@functools.partial(
    pl.kernel,
    mesh=_mesh,
    out_type=jax.ShapeDtypeStruct((BATCH, HIST, EMBED_D), jnp.float32),
    scratch_types=[
        pltpu.VMEM((NUM_ROWS, EMBED_D), jnp.float32),
        pltpu.VMEM((HIST,), jnp.int32),
        pltpu.VMEM((HIST,), jnp.int32),
        pltpu.VMEM((HIST, EMBED_D), jnp.float32),
        pltpu.VMEM((HIST, EMBED_D), jnp.float32),
        pltpu.SemaphoreType.DMA,
        pltpu.SemaphoreType.DMA,
        pltpu.SemaphoreType.DMA,
        pltpu.SemaphoreType.DMA,
    ],
)
def _embed_gather(idx_hbm, table_hbm, out_hbm, table_v, idx0, idx1,
                  rows0, rows1, si0, si1, so0, so1):
    wid = lax.axis_index("s") * _NC + lax.axis_index("c")
    base = wid * _RPW

    pltpu.sync_copy(table_hbm, table_v)

    pltpu.async_copy(idx_hbm.at[base], idx0, si0)
    pltpu.async_copy(idx_hbm.at[base + 1], idx1, si1)

    def expand(iv_ref, rv):
        def grp(g, carry):
            iv = iv_ref[pl.ds(g * _L, _L)]
            for k in range(_L):
                row = iv[k]
                dst = g * _L + k
                for j in range(EMBED_D // _L):
                    rv[dst, pl.ds(j * _L, _L)] = table_v[row,
                                                         pl.ds(j * _L, _L)]
            return carry
        lax.fori_loop(0, _NGRP, grp, 0)
        iv = iv_ref[pl.ds(_TAIL, _L)]
        for k in range(_L):
            row = iv[k]
            for j in range(EMBED_D // _L):
                rv[_TAIL + k, pl.ds(j * _L, _L)] = table_v[row,
                                                           pl.ds(j * _L, _L)]

    def pair(i, carry):
        for b, (iv_ref, rv, si, so) in enumerate((
                (idx0, rows0, si0, so0), (idx1, rows1, si1, so1))):
            r = base + 2 * i + b
            pltpu.make_async_copy(idx_hbm.at[base], iv_ref, si).wait()

            @pl.when(i > 0)
            def _wait_out():
                pltpu.make_async_copy(rv, out_hbm.at[base], so).wait()

            expand(iv_ref, rv)

            @pl.when(r + 2 < base + _RPW)
            def _prefetch_next():
                pltpu.async_copy(idx_hbm.at[r + 2], iv_ref, si)

            pltpu.async_copy(rv, out_hbm.at[r], so)
        return carry

    lax.fori_loop(0, _RPW // 2, pair, 0)

    pltpu.make_async_copy(rows0, out_hbm.at[base], so0).wait()
    pltpu.make_async_copy(rows1, out_hbm.at[base], so1).wait()


def kernel(genres, table):
    return _embed_gather(genres.astype(jnp.int32), table)
